# baseline (speedup 1.0000x reference)
import functools

import jax
import jax.numpy as jnp
from jax import lax
from jax.experimental import pallas as pl
from jax.experimental.pallas import tpu as pltpu
from jax.experimental.pallas import tpu_sc as plsc

N = 10000
E = 320000
D = 128
OUT = 40
OUTP = 128
NPAD = 10240
NW = 32
CHUNK = 128
EPT = 10240
NCH = EPT // CHUNK
EPAD = NW * EPT
TOTAL_CH = EPAD // CHUNK
RPT = NPAD // 16
FAST_C = 0
T_FAST = 120
T_SLOW = 40

@functools.cache
def _mesh():
    return plsc.VectorSubcoreMesh(core_axis_name="c", subcore_axis_name="s")


def _sc_degree(dst3, zerosD, onesD):
    K = 8

    def body(dst3_hbm, zeros_hbm, ones_hbm, out_hbm, dst_v, ones_v, acc_sh,
             semA, semB):
        sems = [semA, semB]
        c = lax.axis_index("c")
        s = lax.axis_index("s")
        w = c * 16 + s
        pltpu.sync_copy(zeros_hbm.at[pl.ds(s * RPT, RPT)],
                        acc_sh.at[pl.ds(s * RPT, RPT)])
        pltpu.sync_copy(ones_hbm, ones_v)
        pltpu.sync_copy(dst3_hbm.at[pl.ds(w * NCH, NCH)], dst_v)
        plsc.subcore_barrier()

        def sg(Gs, _):
            for par in range(2):
                Gr = Gs * 2 + par

                @pl.when(Gr >= 1)
                def _():
                    for _k in range(K):
                        pltpu.make_async_copy(ones_v, acc_sh.at[dst_v.at[0]],
                                              sems[1 - par]).wait()
                for k in range(K):
                    j = Gr * K + k
                    pltpu.async_copy(ones_v, acc_sh.at[dst_v.at[j]],
                                     sems[par], add=True)
            return 0
        lax.fori_loop(0, NCH // (2 * K), sg, 0)
        for _k in range(K):
            pltpu.make_async_copy(ones_v, acc_sh.at[dst_v.at[0]],
                                  sems[1]).wait()
        plsc.subcore_barrier()
        pltpu.sync_copy(acc_sh.at[pl.ds(s * RPT, RPT)],
                        out_hbm.at[c].at[pl.ds(s * RPT, RPT)])

    f = pl.kernel(
        body,
        out_type=jax.ShapeDtypeStruct((2, NPAD, D), jnp.float32),
        mesh=_mesh(),
        scratch_types=[
            pltpu.VMEM((NCH, CHUNK), jnp.int32),
            pltpu.VMEM((CHUNK, D), jnp.float32),
            pltpu.VMEM_SHARED((NPAD, D), jnp.float32),
            pltpu.SemaphoreType.DMA,
            pltpu.SemaphoreType.DMA,
        ],
    )
    return f(dst3, zerosD, onesD)


def _sc_edge_scatter(width):
    IDXC = 8

    def body(src3_hbm, dst3_hbm, g_hbm, zeros_hbm, out_hbm,
             src_v, dst_v, r0, r1, acc_sh, g0, g1, s0, s1):
        rows = [r0, r1]
        gsem = [g0, g1]
        ssem = [s0, s1]
        c = lax.axis_index("c")
        s = lax.axis_index("s")
        nq = jnp.where(c == FAST_C, T_FAST // IDXC, T_SLOW // IDXC)
        start = jnp.where(c == FAST_C, s * T_FAST, 16 * T_FAST + s * T_SLOW)
        pltpu.sync_copy(zeros_hbm.at[pl.ds(s * RPT, RPT)],
                        acc_sh.at[pl.ds(s * RPT, RPT)])
        plsc.subcore_barrier()

        def block(q, _):
            base = start + q * IDXC
            pltpu.sync_copy(src3_hbm.at[pl.ds(base, IDXC)], src_v)
            pltpu.sync_copy(dst3_hbm.at[pl.ds(base, IDXC)], dst_v)
            pltpu.async_copy(g_hbm.at[src_v.at[0]], rows[0], gsem[0])

            def pair(jj, _):
                for bi in range(2):
                    j = jj * 2 + bi

                    @pl.when(j >= 1)
                    def _():
                        pltpu.make_async_copy(rows[1 - bi],
                                              acc_sh.at[dst_v.at[0]],
                                              ssem[1 - bi]).wait()

                    @pl.when(j + 1 < IDXC)
                    def _():
                        pltpu.async_copy(g_hbm.at[src_v.at[j + 1]],
                                         rows[1 - bi], gsem[1 - bi])
                    pltpu.make_async_copy(g_hbm.at[src_v.at[j]], rows[bi],
                                          gsem[bi]).wait()
                    pltpu.async_copy(rows[bi], acc_sh.at[dst_v.at[j]],
                                     ssem[bi], add=True)
                return 0
            lax.fori_loop(0, IDXC // 2, pair, 0)
            pltpu.make_async_copy(rows[(IDXC - 1) % 2], acc_sh.at[dst_v.at[0]],
                                  ssem[(IDXC - 1) % 2]).wait()
            return 0
        lax.fori_loop(0, nq, block, 0)
        plsc.subcore_barrier()
        pltpu.sync_copy(acc_sh.at[pl.ds(s * RPT, RPT)],
                        out_hbm.at[c].at[pl.ds(s * RPT, RPT)])

    return pl.kernel(
        body,
        out_type=jax.ShapeDtypeStruct((2, NPAD, width), jnp.float32),
        mesh=_mesh(),
        scratch_types=[
            pltpu.VMEM((IDXC, CHUNK), jnp.int32),
            pltpu.VMEM((IDXC, CHUNK), jnp.int32),
            pltpu.VMEM((CHUNK, width), jnp.float32),
            pltpu.VMEM((CHUNK, width), jnp.float32),
            pltpu.VMEM_SHARED((NPAD, width), jnp.float32),
        ] + [pltpu.SemaphoreType.DMA] * 4,
    )


BR = 1024
NB = NPAD // BR


def _tc_prep_body(x_ref, w1_ref, wsk_ref, bsk_ref, degp_ref,
                  g1_ref, hsk_ref, dis_ref):
    i = pl.program_id(0)
    deg = degp_ref[0, :, 0] + degp_ref[1, :, 0] + 1.0
    dis = lax.rsqrt(deg)[:, None]
    rows = lax.broadcasted_iota(jnp.int32, (BR, 1), 0) + i * BR
    dis = jnp.where(rows < N, dis, 0.0)
    h1 = jnp.dot(x_ref[...], w1_ref[...], preferred_element_type=jnp.float32)
    g1_ref[...] = dis * h1
    hsk_ref[...] = (
        jnp.dot(x_ref[...], wsk_ref[...], preferred_element_type=jnp.float32)
        + bsk_ref[...])
    dis_ref[...] = jnp.broadcast_to(dis, (BR, 8))


def _tc_prep(xp, W1, Wskip, bskip, degp):
    return pl.pallas_call(
        _tc_prep_body,
        grid=(NB,),
        in_specs=[
            pl.BlockSpec((BR, D), lambda i: (i, 0)),
            pl.BlockSpec((D, D), lambda i: (0, 0)),
            pl.BlockSpec((D, D), lambda i: (0, 0)),
            pl.BlockSpec((1, D), lambda i: (0, 0)),
            pl.BlockSpec((2, BR, D), lambda i: (0, i, 0)),
        ],
        out_specs=[
            pl.BlockSpec((BR, D), lambda i: (i, 0)),
            pl.BlockSpec((BR, D), lambda i: (i, 0)),
            pl.BlockSpec((BR, 8), lambda i: (i, 0)),
        ],
        out_shape=[
            jax.ShapeDtypeStruct((NPAD, D), jnp.float32),
            jax.ShapeDtypeStruct((NPAD, D), jnp.float32),
            jax.ShapeDtypeStruct((NPAD, 8), jnp.float32),
        ],
    )(xp, W1, Wskip, bskip, degp)


def _tc_conv1_body(accp_ref, g1_ref, dis_ref, b1_ref, out1_ref, stats_ref):
    i = pl.program_id(0)
    s1 = accp_ref[0] + accp_ref[1] + g1_ref[...]
    out1 = dis_ref[:, 0:1] * s1 + b1_ref[...]
    out1_ref[...] = out1
    rows = lax.broadcasted_iota(jnp.int32, (BR, 1), 0) + i * BR
    v = jnp.where(rows < N, out1, 0.0)

    @pl.when(i == 0)
    def _():
        stats_ref[...] = jnp.zeros_like(stats_ref)
    stats_ref[0, :] += jnp.sum(v, axis=0)
    stats_ref[1, :] += jnp.sum(v * v, axis=0)


def _tc_conv1(accp, g1, dis8, b1):
    return pl.pallas_call(
        _tc_conv1_body,
        grid=(NB,),
        in_specs=[
            pl.BlockSpec((2, BR, D), lambda i: (0, i, 0)),
            pl.BlockSpec((BR, D), lambda i: (i, 0)),
            pl.BlockSpec((BR, 8), lambda i: (i, 0)),
            pl.BlockSpec((1, D), lambda i: (0, 0)),
        ],
        out_specs=[
            pl.BlockSpec((BR, D), lambda i: (i, 0)),
            pl.BlockSpec((2, D), lambda i: (0, 0)),
        ],
        out_shape=[
            jax.ShapeDtypeStruct((NPAD, D), jnp.float32),
            jax.ShapeDtypeStruct((2, D), jnp.float32),
        ],
    )(accp, g1, dis8, b1)


def _tc_mid_body(out1_ref, stats_ref, gam_ref, bet_ref, hsk_ref, w2_ref,
                 dis_ref, g2_ref):
    mean = stats_ref[0, :] / N
    var = stats_ref[1, :] / N - mean * mean
    inv = gam_ref[0, :] * lax.rsqrt(var + 1e-5)
    bn = (out1_ref[...] - mean[None, :]) * inv[None, :] + bet_ref[...]
    h = jnp.maximum(bn, 0.0) + hsk_ref[...]
    z2 = jnp.dot(h, w2_ref[...], preferred_element_type=jnp.float32)
    g2_ref[...] = dis_ref[:, 0:1] * z2


def _tc_mid(out1, stats, gamma1, beta1, hsk, W2p, dis8):
    return pl.pallas_call(
        _tc_mid_body,
        grid=(NB,),
        in_specs=[
            pl.BlockSpec((BR, D), lambda i: (i, 0)),
            pl.BlockSpec((2, D), lambda i: (0, 0)),
            pl.BlockSpec((1, D), lambda i: (0, 0)),
            pl.BlockSpec((1, D), lambda i: (0, 0)),
            pl.BlockSpec((BR, D), lambda i: (i, 0)),
            pl.BlockSpec((D, OUTP), lambda i: (0, 0)),
            pl.BlockSpec((BR, 8), lambda i: (i, 0)),
        ],
        out_specs=pl.BlockSpec((BR, OUTP), lambda i: (i, 0)),
        out_shape=jax.ShapeDtypeStruct((NPAD, OUTP), jnp.float32),
    )(out1, stats, gamma1, beta1, hsk, W2p, dis8)


def _tc_final_body(acc2_ref, g2_ref, dis_ref, b2_ref, o_ref):
    s2 = acc2_ref[0] + acc2_ref[1] + g2_ref[...]
    o = dis_ref[:, 0:1] * s2 + b2_ref[...]
    col = lax.broadcasted_iota(jnp.int32, (BR, OUTP), 1)
    valid = col < OUT
    om = jnp.where(valid, o, -1e30)
    mx = jnp.max(om, axis=1, keepdims=True)
    e = jnp.where(valid, jnp.exp(om - mx), 0.0)
    lse = jnp.log(jnp.sum(e, axis=1, keepdims=True))
    o_ref[...] = om - mx - lse


def _tc_final(acc2, g2, dis8, b2p):
    return pl.pallas_call(
        _tc_final_body,
        grid=(NB,),
        in_specs=[
            pl.BlockSpec((2, BR, OUTP), lambda i: (0, i, 0)),
            pl.BlockSpec((BR, OUTP), lambda i: (i, 0)),
            pl.BlockSpec((BR, 8), lambda i: (i, 0)),
            pl.BlockSpec((1, OUTP), lambda i: (0, 0)),
        ],
        out_specs=pl.BlockSpec((BR, OUTP), lambda i: (i, 0)),
        out_shape=jax.ShapeDtypeStruct((NPAD, OUTP), jnp.float32),
    )(acc2, g2, dis8, b2p)


def kernel(x, edge_index, W1, b1, gamma1, beta1, Wskip, bskip, W2, b2):
    xp = jnp.zeros((NPAD, D), jnp.float32).at[:N].set(x)
    src = edge_index[0]
    dst = edge_index[1]
    pad = jnp.full((EPAD - E,), N, jnp.int32)
    src3 = jnp.concatenate([src, pad]).reshape(TOTAL_CH, CHUNK)
    dst3 = jnp.concatenate([dst, pad]).reshape(TOTAL_CH, CHUNK)

    onesD = jnp.ones((CHUNK, D), jnp.float32)
    zerosD = jnp.zeros((NPAD, D), jnp.float32)
    zerosP = jnp.zeros((NPAD, OUTP), jnp.float32)
    W2p = jnp.zeros((D, OUTP), jnp.float32).at[:, :OUT].set(W2)
    b2p = jnp.zeros((1, OUTP), jnp.float32).at[0, :OUT].set(b2)

    degp = _sc_degree(dst3, zerosD, onesD)
    g1, hsk, dis8 = _tc_prep(xp, W1, Wskip, bskip.reshape(1, D), degp)
    accp = _sc_edge_scatter(D)(src3, dst3, g1, zerosD)
    out1, stats = _tc_conv1(accp, g1, dis8, b1.reshape(1, D))
    g2 = _tc_mid(out1, stats, gamma1.reshape(1, D), beta1.reshape(1, D),
                 hsk, W2p, dis8)
    acc2 = _sc_edge_scatter(OUTP)(src3, dst3, g2, zerosP)
    o = _tc_final(acc2, g2, dis8, b2p)
    return o[:N, :OUT]


# speedup vs baseline: 1.0511x; 1.0511x over previous
import functools

import jax
import jax.numpy as jnp
from jax import lax
from jax.experimental import pallas as pl
from jax.experimental.pallas import tpu as pltpu
from jax.experimental.pallas import tpu_sc as plsc

N = 10000
E = 320000
D = 128
OUT = 40
OUTP = 128
NPAD = 10240
NW = 32
CHUNK = 128
EPT = 10240
NCH = EPT // CHUNK
EPAD = NW * EPT
TOTAL_CH = EPAD // CHUNK
RPT = NPAD // 16
FAST_C = 1
T_FAST = 120
T_SLOW = 40

@functools.cache
def _mesh():
    return plsc.VectorSubcoreMesh(core_axis_name="c", subcore_axis_name="s")


def _sc_degree(dst3, zerosD, onesD):
    K = 8

    def body(dst3_hbm, zeros_hbm, ones_hbm, out_hbm, dst_v, ones_v, acc_sh,
             semA, semB):
        sems = [semA, semB]
        c = lax.axis_index("c")
        s = lax.axis_index("s")
        w = c * 16 + s
        pltpu.sync_copy(zeros_hbm.at[pl.ds(s * RPT, RPT)],
                        acc_sh.at[pl.ds(s * RPT, RPT)])
        pltpu.sync_copy(ones_hbm, ones_v)
        pltpu.sync_copy(dst3_hbm.at[pl.ds(w * NCH, NCH)], dst_v)
        plsc.subcore_barrier()

        def sg(Gs, _):
            for par in range(2):
                Gr = Gs * 2 + par

                @pl.when(Gr >= 1)
                def _():
                    for _k in range(K):
                        pltpu.make_async_copy(ones_v, acc_sh.at[dst_v.at[0]],
                                              sems[1 - par]).wait()
                for k in range(K):
                    j = Gr * K + k
                    pltpu.async_copy(ones_v, acc_sh.at[dst_v.at[j]],
                                     sems[par], add=True)
            return 0
        lax.fori_loop(0, NCH // (2 * K), sg, 0)
        for _k in range(K):
            pltpu.make_async_copy(ones_v, acc_sh.at[dst_v.at[0]],
                                  sems[1]).wait()
        plsc.subcore_barrier()
        pltpu.sync_copy(acc_sh.at[pl.ds(s * RPT, RPT)],
                        out_hbm.at[c].at[pl.ds(s * RPT, RPT)])

    f = pl.kernel(
        body,
        out_type=jax.ShapeDtypeStruct((2, NPAD, D), jnp.float32),
        mesh=_mesh(),
        scratch_types=[
            pltpu.VMEM((NCH, CHUNK), jnp.int32),
            pltpu.VMEM((CHUNK, D), jnp.float32),
            pltpu.VMEM_SHARED((NPAD, D), jnp.float32),
            pltpu.SemaphoreType.DMA,
            pltpu.SemaphoreType.DMA,
        ],
    )
    return f(dst3, zerosD, onesD)


def _sc_edge_scatter(width):
    IDXC = 8

    def body(src3_hbm, dst3_hbm, g_hbm, zeros_hbm, out_hbm,
             src_v, dst_v, r0, r1, acc_sh, g0, g1, s0, s1):
        rows = [r0, r1]
        gsem = [g0, g1]
        ssem = [s0, s1]
        c = lax.axis_index("c")
        s = lax.axis_index("s")
        nq = jnp.where(c == FAST_C, T_FAST // IDXC, T_SLOW // IDXC)
        start = jnp.where(c == FAST_C, s * T_FAST, 16 * T_FAST + s * T_SLOW)
        pltpu.sync_copy(zeros_hbm.at[pl.ds(s * RPT, RPT)],
                        acc_sh.at[pl.ds(s * RPT, RPT)])
        plsc.subcore_barrier()

        def block(q, _):
            base = start + q * IDXC
            pltpu.sync_copy(src3_hbm.at[pl.ds(base, IDXC)], src_v)
            pltpu.sync_copy(dst3_hbm.at[pl.ds(base, IDXC)], dst_v)
            pltpu.async_copy(g_hbm.at[src_v.at[0]], rows[0], gsem[0])

            def pair(jj, _):
                for bi in range(2):
                    j = jj * 2 + bi

                    @pl.when(j >= 1)
                    def _():
                        pltpu.make_async_copy(rows[1 - bi],
                                              acc_sh.at[dst_v.at[0]],
                                              ssem[1 - bi]).wait()

                    @pl.when(j + 1 < IDXC)
                    def _():
                        pltpu.async_copy(g_hbm.at[src_v.at[j + 1]],
                                         rows[1 - bi], gsem[1 - bi])
                    pltpu.make_async_copy(g_hbm.at[src_v.at[j]], rows[bi],
                                          gsem[bi]).wait()
                    pltpu.async_copy(rows[bi], acc_sh.at[dst_v.at[j]],
                                     ssem[bi], add=True)
                return 0
            lax.fori_loop(0, IDXC // 2, pair, 0)
            pltpu.make_async_copy(rows[(IDXC - 1) % 2], acc_sh.at[dst_v.at[0]],
                                  ssem[(IDXC - 1) % 2]).wait()
            return 0
        lax.fori_loop(0, nq, block, 0)
        plsc.subcore_barrier()
        pltpu.sync_copy(acc_sh.at[pl.ds(s * RPT, RPT)],
                        out_hbm.at[c].at[pl.ds(s * RPT, RPT)])

    return pl.kernel(
        body,
        out_type=jax.ShapeDtypeStruct((2, NPAD, width), jnp.float32),
        mesh=_mesh(),
        scratch_types=[
            pltpu.VMEM((IDXC, CHUNK), jnp.int32),
            pltpu.VMEM((IDXC, CHUNK), jnp.int32),
            pltpu.VMEM((CHUNK, width), jnp.float32),
            pltpu.VMEM((CHUNK, width), jnp.float32),
            pltpu.VMEM_SHARED((NPAD, width), jnp.float32),
        ] + [pltpu.SemaphoreType.DMA] * 4,
    )


BR = 1024
NB = NPAD // BR


def _tc_prep_body(x_ref, w1_ref, wsk_ref, bsk_ref, degp_ref,
                  g1_ref, hsk_ref, dis_ref):
    i = pl.program_id(0)
    deg = degp_ref[0, :, 0] + degp_ref[1, :, 0] + 1.0
    dis = lax.rsqrt(deg)[:, None]
    rows = lax.broadcasted_iota(jnp.int32, (BR, 1), 0) + i * BR
    dis = jnp.where(rows < N, dis, 0.0)
    h1 = jnp.dot(x_ref[...], w1_ref[...], preferred_element_type=jnp.float32)
    g1_ref[...] = dis * h1
    hsk_ref[...] = (
        jnp.dot(x_ref[...], wsk_ref[...], preferred_element_type=jnp.float32)
        + bsk_ref[...])
    dis_ref[...] = jnp.broadcast_to(dis, (BR, 8))


def _tc_prep(xp, W1, Wskip, bskip, degp):
    return pl.pallas_call(
        _tc_prep_body,
        grid=(NB,),
        in_specs=[
            pl.BlockSpec((BR, D), lambda i: (i, 0)),
            pl.BlockSpec((D, D), lambda i: (0, 0)),
            pl.BlockSpec((D, D), lambda i: (0, 0)),
            pl.BlockSpec((1, D), lambda i: (0, 0)),
            pl.BlockSpec((2, BR, D), lambda i: (0, i, 0)),
        ],
        out_specs=[
            pl.BlockSpec((BR, D), lambda i: (i, 0)),
            pl.BlockSpec((BR, D), lambda i: (i, 0)),
            pl.BlockSpec((BR, 8), lambda i: (i, 0)),
        ],
        out_shape=[
            jax.ShapeDtypeStruct((NPAD, D), jnp.float32),
            jax.ShapeDtypeStruct((NPAD, D), jnp.float32),
            jax.ShapeDtypeStruct((NPAD, 8), jnp.float32),
        ],
    )(xp, W1, Wskip, bskip, degp)


def _tc_conv1_body(accp_ref, g1_ref, dis_ref, b1_ref, out1_ref, stats_ref):
    i = pl.program_id(0)
    s1 = accp_ref[0] + accp_ref[1] + g1_ref[...]
    out1 = dis_ref[:, 0:1] * s1 + b1_ref[...]
    out1_ref[...] = out1
    rows = lax.broadcasted_iota(jnp.int32, (BR, 1), 0) + i * BR
    v = jnp.where(rows < N, out1, 0.0)

    @pl.when(i == 0)
    def _():
        stats_ref[...] = jnp.zeros_like(stats_ref)
    stats_ref[0, :] += jnp.sum(v, axis=0)
    stats_ref[1, :] += jnp.sum(v * v, axis=0)


def _tc_conv1(accp, g1, dis8, b1):
    return pl.pallas_call(
        _tc_conv1_body,
        grid=(NB,),
        in_specs=[
            pl.BlockSpec((2, BR, D), lambda i: (0, i, 0)),
            pl.BlockSpec((BR, D), lambda i: (i, 0)),
            pl.BlockSpec((BR, 8), lambda i: (i, 0)),
            pl.BlockSpec((1, D), lambda i: (0, 0)),
        ],
        out_specs=[
            pl.BlockSpec((BR, D), lambda i: (i, 0)),
            pl.BlockSpec((2, D), lambda i: (0, 0)),
        ],
        out_shape=[
            jax.ShapeDtypeStruct((NPAD, D), jnp.float32),
            jax.ShapeDtypeStruct((2, D), jnp.float32),
        ],
    )(accp, g1, dis8, b1)


def _tc_mid_body(out1_ref, stats_ref, gam_ref, bet_ref, hsk_ref, w2_ref,
                 dis_ref, g2_ref):
    mean = stats_ref[0, :] / N
    var = stats_ref[1, :] / N - mean * mean
    inv = gam_ref[0, :] * lax.rsqrt(var + 1e-5)
    bn = (out1_ref[...] - mean[None, :]) * inv[None, :] + bet_ref[...]
    h = jnp.maximum(bn, 0.0) + hsk_ref[...]
    z2 = jnp.dot(h, w2_ref[...], preferred_element_type=jnp.float32)
    g2_ref[...] = dis_ref[:, 0:1] * z2


def _tc_mid(out1, stats, gamma1, beta1, hsk, W2p, dis8):
    return pl.pallas_call(
        _tc_mid_body,
        grid=(NB,),
        in_specs=[
            pl.BlockSpec((BR, D), lambda i: (i, 0)),
            pl.BlockSpec((2, D), lambda i: (0, 0)),
            pl.BlockSpec((1, D), lambda i: (0, 0)),
            pl.BlockSpec((1, D), lambda i: (0, 0)),
            pl.BlockSpec((BR, D), lambda i: (i, 0)),
            pl.BlockSpec((D, OUTP), lambda i: (0, 0)),
            pl.BlockSpec((BR, 8), lambda i: (i, 0)),
        ],
        out_specs=pl.BlockSpec((BR, OUTP), lambda i: (i, 0)),
        out_shape=jax.ShapeDtypeStruct((NPAD, OUTP), jnp.float32),
    )(out1, stats, gamma1, beta1, hsk, W2p, dis8)


def _tc_final_body(acc2_ref, g2_ref, dis_ref, b2_ref, o_ref):
    s2 = acc2_ref[0] + acc2_ref[1] + g2_ref[...]
    o = dis_ref[:, 0:1] * s2 + b2_ref[...]
    col = lax.broadcasted_iota(jnp.int32, (BR, OUTP), 1)
    valid = col < OUT
    om = jnp.where(valid, o, -1e30)
    mx = jnp.max(om, axis=1, keepdims=True)
    e = jnp.where(valid, jnp.exp(om - mx), 0.0)
    lse = jnp.log(jnp.sum(e, axis=1, keepdims=True))
    o_ref[...] = om - mx - lse


def _tc_final(acc2, g2, dis8, b2p):
    return pl.pallas_call(
        _tc_final_body,
        grid=(NB,),
        in_specs=[
            pl.BlockSpec((2, BR, OUTP), lambda i: (0, i, 0)),
            pl.BlockSpec((BR, OUTP), lambda i: (i, 0)),
            pl.BlockSpec((BR, 8), lambda i: (i, 0)),
            pl.BlockSpec((1, OUTP), lambda i: (0, 0)),
        ],
        out_specs=pl.BlockSpec((BR, OUTP), lambda i: (i, 0)),
        out_shape=jax.ShapeDtypeStruct((NPAD, OUTP), jnp.float32),
    )(acc2, g2, dis8, b2p)


def kernel(x, edge_index, W1, b1, gamma1, beta1, Wskip, bskip, W2, b2):
    xp = jnp.zeros((NPAD, D), jnp.float32).at[:N].set(x)
    src = edge_index[0]
    dst = edge_index[1]
    pad = jnp.full((EPAD - E,), N, jnp.int32)
    src3 = jnp.concatenate([src, pad]).reshape(TOTAL_CH, CHUNK)
    dst3 = jnp.concatenate([dst, pad]).reshape(TOTAL_CH, CHUNK)

    onesD = jnp.ones((CHUNK, D), jnp.float32)
    zerosD = jnp.zeros((NPAD, D), jnp.float32)
    zerosP = jnp.zeros((NPAD, OUTP), jnp.float32)
    W2p = jnp.zeros((D, OUTP), jnp.float32).at[:, :OUT].set(W2)
    b2p = jnp.zeros((1, OUTP), jnp.float32).at[0, :OUT].set(b2)

    degp = _sc_degree(dst3, zerosD, onesD)
    g1, hsk, dis8 = _tc_prep(xp, W1, Wskip, bskip.reshape(1, D), degp)
    accp = _sc_edge_scatter(D)(src3, dst3, g1, zerosD)
    out1, stats = _tc_conv1(accp, g1, dis8, b1.reshape(1, D))
    g2 = _tc_mid(out1, stats, gamma1.reshape(1, D), beta1.reshape(1, D),
                 hsk, W2p, dis8)
    acc2 = _sc_edge_scatter(OUTP)(src3, dst3, g2, zerosP)
    o = _tc_final(acc2, g2, dis8, b2p)
    return o[:N, :OUT]


# speedup vs baseline: 2.5931x; 2.4671x over previous
import functools

import jax
import jax.numpy as jnp
from jax import lax
from jax.experimental import pallas as pl
from jax.experimental.pallas import tpu as pltpu
from jax.experimental.pallas import tpu_sc as plsc

N = 10000
E = 320000
D = 128
OUT = 40
OUTP = 128
NPAD = 10240
NW = 32
CHUNK = 128
EPT = 10240
NCH = EPT // CHUNK
EPAD = NW * EPT
TOTAL_CH = EPAD // CHUNK
RPT = NPAD // 16

@functools.cache
def _mesh():
    return plsc.VectorSubcoreMesh(core_axis_name="c", subcore_axis_name="s")


def _sc_degree(dst3, zerosD, onesD):
    K = 8

    def body(dst3_hbm, zeros_hbm, ones_hbm, out_hbm, dst_v, ones_v, acc_sh,
             semA, semB):
        sems = [semA, semB]
        c = lax.axis_index("c")
        s = lax.axis_index("s")
        w = c * 16 + s
        pltpu.sync_copy(zeros_hbm.at[pl.ds(s * RPT, RPT)],
                        acc_sh.at[pl.ds(s * RPT, RPT)])
        pltpu.sync_copy(ones_hbm, ones_v)
        pltpu.sync_copy(dst3_hbm.at[pl.ds(w * NCH, NCH)], dst_v)
        plsc.subcore_barrier()

        def sg(Gs, _):
            for par in range(2):
                Gr = Gs * 2 + par

                @pl.when(Gr >= 1)
                def _():
                    for _k in range(K):
                        pltpu.make_async_copy(ones_v, acc_sh.at[dst_v.at[0]],
                                              sems[1 - par]).wait()
                for k in range(K):
                    j = Gr * K + k
                    pltpu.async_copy(ones_v, acc_sh.at[dst_v.at[j]],
                                     sems[par], add=True)
            return 0
        lax.fori_loop(0, NCH // (2 * K), sg, 0)
        for _k in range(K):
            pltpu.make_async_copy(ones_v, acc_sh.at[dst_v.at[0]],
                                  sems[1]).wait()
        plsc.subcore_barrier()
        pltpu.sync_copy(acc_sh.at[pl.ds(s * RPT, RPT)],
                        out_hbm.at[c].at[pl.ds(s * RPT, RPT)])

    f = pl.kernel(
        body,
        out_type=jax.ShapeDtypeStruct((2, NPAD, D), jnp.float32),
        mesh=_mesh(),
        scratch_types=[
            pltpu.VMEM((NCH, CHUNK), jnp.int32),
            pltpu.VMEM((CHUNK, D), jnp.float32),
            pltpu.VMEM_SHARED((NPAD, D), jnp.float32),
            pltpu.SemaphoreType.DMA,
            pltpu.SemaphoreType.DMA,
        ],
    )
    return f(dst3, zerosD, onesD)


def _sc_edge_scatter(width):
    IDXC = 8

    def body(src3_hbm, dst3_hbm, g_hbm, zeros_hbm, out_hbm,
             src_v, dst_v, r0, r1, acc_sh, g0, g1, s0, s1):
        rows = [r0, r1]
        gsem = [g0, g1]
        ssem = [s0, s1]
        c = lax.axis_index("c")
        s = lax.axis_index("s")
        start = (c * 16 + s) * NCH
        pltpu.sync_copy(zeros_hbm.at[pl.ds(s * RPT, RPT)],
                        acc_sh.at[pl.ds(s * RPT, RPT)])
        plsc.subcore_barrier()

        def block(q, _):
            base = start + q * IDXC
            pltpu.sync_copy(src3_hbm.at[pl.ds(base, IDXC)], src_v)
            pltpu.sync_copy(dst3_hbm.at[pl.ds(base, IDXC)], dst_v)
            pltpu.async_copy(g_hbm.at[src_v.at[0]], rows[0], gsem[0])

            def pair(jj, _):
                for bi in range(2):
                    j = jj * 2 + bi

                    @pl.when(j >= 1)
                    def _():
                        pltpu.make_async_copy(rows[1 - bi],
                                              acc_sh.at[dst_v.at[0]],
                                              ssem[1 - bi]).wait()

                    @pl.when(j + 1 < IDXC)
                    def _():
                        pltpu.async_copy(g_hbm.at[src_v.at[j + 1]],
                                         rows[1 - bi], gsem[1 - bi])
                    pltpu.make_async_copy(g_hbm.at[src_v.at[j]], rows[bi],
                                          gsem[bi]).wait()
                    pltpu.async_copy(rows[bi], acc_sh.at[dst_v.at[j]],
                                     ssem[bi], add=True)
                return 0
            lax.fori_loop(0, IDXC // 2, pair, 0)
            pltpu.make_async_copy(rows[(IDXC - 1) % 2], acc_sh.at[dst_v.at[0]],
                                  ssem[(IDXC - 1) % 2]).wait()
            return 0
        lax.fori_loop(0, NCH // IDXC, block, 0)
        plsc.subcore_barrier()
        pltpu.sync_copy(acc_sh.at[pl.ds(s * RPT, RPT)],
                        out_hbm.at[c].at[pl.ds(s * RPT, RPT)])

    return pl.kernel(
        body,
        out_type=jax.ShapeDtypeStruct((2, NPAD, width), jnp.float32),
        mesh=_mesh(),
        scratch_types=[
            pltpu.VMEM((IDXC, CHUNK), jnp.int32),
            pltpu.VMEM((IDXC, CHUNK), jnp.int32),
            pltpu.VMEM((CHUNK, width), jnp.float32),
            pltpu.VMEM((CHUNK, width), jnp.float32),
            pltpu.VMEM_SHARED((NPAD, width), jnp.float32),
        ] + [pltpu.SemaphoreType.DMA] * 4,
    )


BR = 1024
NB = NPAD // BR


def _tc_prep_body(x_ref, w1_ref, wsk_ref, bsk_ref, degp_ref,
                  g1_ref, hsk_ref, dis_ref):
    i = pl.program_id(0)
    deg = degp_ref[0, :, 0] + degp_ref[1, :, 0] + 1.0
    dis = lax.rsqrt(deg)[:, None]
    rows = lax.broadcasted_iota(jnp.int32, (BR, 1), 0) + i * BR
    dis = jnp.where(rows < N, dis, 0.0)
    h1 = jnp.dot(x_ref[...], w1_ref[...], preferred_element_type=jnp.float32)
    g1_ref[...] = dis * h1
    hsk_ref[...] = (
        jnp.dot(x_ref[...], wsk_ref[...], preferred_element_type=jnp.float32)
        + bsk_ref[...])
    dis_ref[...] = jnp.broadcast_to(dis, (BR, 8))


def _tc_prep(xp, W1, Wskip, bskip, degp):
    return pl.pallas_call(
        _tc_prep_body,
        grid=(NB,),
        in_specs=[
            pl.BlockSpec((BR, D), lambda i: (i, 0)),
            pl.BlockSpec((D, D), lambda i: (0, 0)),
            pl.BlockSpec((D, D), lambda i: (0, 0)),
            pl.BlockSpec((1, D), lambda i: (0, 0)),
            pl.BlockSpec((2, BR, D), lambda i: (0, i, 0)),
        ],
        out_specs=[
            pl.BlockSpec((BR, D), lambda i: (i, 0)),
            pl.BlockSpec((BR, D), lambda i: (i, 0)),
            pl.BlockSpec((BR, 8), lambda i: (i, 0)),
        ],
        out_shape=[
            jax.ShapeDtypeStruct((NPAD, D), jnp.float32),
            jax.ShapeDtypeStruct((NPAD, D), jnp.float32),
            jax.ShapeDtypeStruct((NPAD, 8), jnp.float32),
        ],
    )(xp, W1, Wskip, bskip, degp)


def _tc_conv1_body(accp_ref, g1_ref, dis_ref, b1_ref, out1_ref, stats_ref):
    i = pl.program_id(0)
    s1 = accp_ref[0] + accp_ref[1] + g1_ref[...]
    out1 = dis_ref[:, 0:1] * s1 + b1_ref[...]
    out1_ref[...] = out1
    rows = lax.broadcasted_iota(jnp.int32, (BR, 1), 0) + i * BR
    v = jnp.where(rows < N, out1, 0.0)

    @pl.when(i == 0)
    def _():
        stats_ref[...] = jnp.zeros_like(stats_ref)
    stats_ref[0, :] += jnp.sum(v, axis=0)
    stats_ref[1, :] += jnp.sum(v * v, axis=0)


def _tc_conv1(accp, g1, dis8, b1):
    return pl.pallas_call(
        _tc_conv1_body,
        grid=(NB,),
        in_specs=[
            pl.BlockSpec((2, BR, D), lambda i: (0, i, 0)),
            pl.BlockSpec((BR, D), lambda i: (i, 0)),
            pl.BlockSpec((BR, 8), lambda i: (i, 0)),
            pl.BlockSpec((1, D), lambda i: (0, 0)),
        ],
        out_specs=[
            pl.BlockSpec((BR, D), lambda i: (i, 0)),
            pl.BlockSpec((2, D), lambda i: (0, 0)),
        ],
        out_shape=[
            jax.ShapeDtypeStruct((NPAD, D), jnp.float32),
            jax.ShapeDtypeStruct((2, D), jnp.float32),
        ],
    )(accp, g1, dis8, b1)


def _tc_mid_body(out1_ref, stats_ref, gam_ref, bet_ref, hsk_ref, w2_ref,
                 dis_ref, g2_ref):
    mean = stats_ref[0, :] / N
    var = stats_ref[1, :] / N - mean * mean
    inv = gam_ref[0, :] * lax.rsqrt(var + 1e-5)
    bn = (out1_ref[...] - mean[None, :]) * inv[None, :] + bet_ref[...]
    h = jnp.maximum(bn, 0.0) + hsk_ref[...]
    z2 = jnp.dot(h, w2_ref[...], preferred_element_type=jnp.float32)
    g2_ref[...] = dis_ref[:, 0:1] * z2


def _tc_mid(out1, stats, gamma1, beta1, hsk, W2p, dis8):
    return pl.pallas_call(
        _tc_mid_body,
        grid=(NB,),
        in_specs=[
            pl.BlockSpec((BR, D), lambda i: (i, 0)),
            pl.BlockSpec((2, D), lambda i: (0, 0)),
            pl.BlockSpec((1, D), lambda i: (0, 0)),
            pl.BlockSpec((1, D), lambda i: (0, 0)),
            pl.BlockSpec((BR, D), lambda i: (i, 0)),
            pl.BlockSpec((D, OUTP), lambda i: (0, 0)),
            pl.BlockSpec((BR, 8), lambda i: (i, 0)),
        ],
        out_specs=pl.BlockSpec((BR, OUTP), lambda i: (i, 0)),
        out_shape=jax.ShapeDtypeStruct((NPAD, OUTP), jnp.float32),
    )(out1, stats, gamma1, beta1, hsk, W2p, dis8)


def _tc_final_body(acc2_ref, g2_ref, dis_ref, b2_ref, o_ref):
    s2 = acc2_ref[0] + acc2_ref[1] + g2_ref[...]
    o = dis_ref[:, 0:1] * s2 + b2_ref[...]
    col = lax.broadcasted_iota(jnp.int32, (BR, OUTP), 1)
    valid = col < OUT
    om = jnp.where(valid, o, -1e30)
    mx = jnp.max(om, axis=1, keepdims=True)
    e = jnp.where(valid, jnp.exp(om - mx), 0.0)
    lse = jnp.log(jnp.sum(e, axis=1, keepdims=True))
    o_ref[...] = om - mx - lse


def _tc_final(acc2, g2, dis8, b2p):
    return pl.pallas_call(
        _tc_final_body,
        grid=(NB,),
        in_specs=[
            pl.BlockSpec((2, BR, OUTP), lambda i: (0, i, 0)),
            pl.BlockSpec((BR, OUTP), lambda i: (i, 0)),
            pl.BlockSpec((BR, 8), lambda i: (i, 0)),
            pl.BlockSpec((1, OUTP), lambda i: (0, 0)),
        ],
        out_specs=pl.BlockSpec((BR, OUTP), lambda i: (i, 0)),
        out_shape=jax.ShapeDtypeStruct((NPAD, OUTP), jnp.float32),
    )(acc2, g2, dis8, b2p)


def kernel(x, edge_index, W1, b1, gamma1, beta1, Wskip, bskip, W2, b2):
    xp = jnp.zeros((NPAD, D), jnp.float32).at[:N].set(x)
    src = edge_index[0]
    dst = edge_index[1]
    pad = N + (jnp.arange(EPAD - E, dtype=jnp.int32) % (NPAD - N))
    src3 = jnp.concatenate([src, pad]).reshape(TOTAL_CH, CHUNK)
    dst3 = jnp.concatenate([dst, pad]).reshape(TOTAL_CH, CHUNK)

    onesD = jnp.ones((CHUNK, D), jnp.float32)
    zerosD = jnp.zeros((NPAD, D), jnp.float32)
    zerosP = jnp.zeros((NPAD, OUTP), jnp.float32)
    W2p = jnp.zeros((D, OUTP), jnp.float32).at[:, :OUT].set(W2)
    b2p = jnp.zeros((1, OUTP), jnp.float32).at[0, :OUT].set(b2)

    degp = _sc_degree(dst3, zerosD, onesD)
    g1, hsk, dis8 = _tc_prep(xp, W1, Wskip, bskip.reshape(1, D), degp)
    accp = _sc_edge_scatter(D)(src3, dst3, g1, zerosD)
    out1, stats = _tc_conv1(accp, g1, dis8, b1.reshape(1, D))
    g2 = _tc_mid(out1, stats, gamma1.reshape(1, D), beta1.reshape(1, D),
                 hsk, W2p, dis8)
    acc2 = _sc_edge_scatter(OUTP)(src3, dst3, g2, zerosP)
    o = _tc_final(acc2, g2, dis8, b2p)
    return o[:N, :OUT]
